# baseline (speedup 1.0000x reference)
import functools

import jax
import jax.numpy as jnp
from jax import lax
from jax.experimental import pallas as pl
from jax.experimental.pallas import tpu as pltpu
from jax.experimental.pallas import tpu_sc as plsc

_D = 64
_SCALE = float(_D) ** 0.5
_NW = 32
_CB = 128
_NBUF = 4
_L16 = 16


@functools.lru_cache(maxsize=None)
def _emb_kernel(n_b, n_l):
    b_per_w = n_b // _NW
    blk_per_w = b_per_w // _CB
    n_chunks = n_l * blk_per_w
    n_groups = n_chunks // _NBUF
    assert n_chunks % _NBUF == 0 and n_groups >= 3
    mesh = plsc.VectorSubcoreMesh(core_axis_name="c", subcore_axis_name="s")

    @functools.partial(
        pl.kernel,
        mesh=mesh,
        out_type=jax.ShapeDtypeStruct((n_l, _D, n_b), jnp.float32),
        scratch_types=[
            pltpu.VMEM((_NBUF, _CB), jnp.int32),
            pltpu.VMEM((_NBUF, _CB), jnp.int32),
            pltpu.VMEM((_NBUF, _CB), jnp.int32),
            pltpu.VMEM((_NBUF, _CB, 2 * _D), jnp.float32),
            pltpu.VMEM((_NBUF, _D, _CB), jnp.float32),
            pltpu.SemaphoreType.DMA((_NBUF,)),
            pltpu.SemaphoreType.DMA((_NBUF,)),
            pltpu.SemaphoreType.DMA((_NBUF,)),
        ],
        compiler_params=pltpu.CompilerParams(
            use_tc_tiling_on_sc=True, needs_layout_passes=False
        ),
    )
    def k(event_hbm, table_hbm, out_hbm, idx_v, row_v, col_v, rows_v, tr_v,
          sem_i, sem_g, sem_o):
        wid = lax.axis_index("s") * 2 + lax.axis_index("c")
        b0w = wid * b_per_w

        def chunk_lb(g):
            return g // blk_per_w, b0w + (g % blk_per_w) * _CB

        def launch_idx(g, b):
            l, cb0 = chunk_lb(g)
            pltpu.async_copy(
                event_hbm.at[l, pl.ds(cb0, _CB)], idx_v.at[b], sem_i.at[b]
            )

        def launch_gather(b):
            for v in range(_CB // _L16):
                sl = pl.ds(v * _L16, _L16)
                iv = idx_v.at[b][sl]
                row_v.at[b][sl] = lax.shift_right_logical(iv, 1)
                col_v.at[b][sl] = lax.shift_left(
                    lax.bitwise_and(iv, 1), 6
                )
            pltpu.async_copy(
                table_hbm.at[row_v.at[b]], rows_v.at[b], sem_g.at[b]
            )

        def wait_idx(b):
            pltpu.make_async_copy(
                event_hbm.at[0, pl.ds(0, _CB)], idx_v.at[b], sem_i.at[b]
            ).wait()

        def wait_gather(b):
            pltpu.make_async_copy(
                table_hbm.at[pl.ds(0, _CB)], rows_v.at[b], sem_g.at[b]
            ).wait()

        def wait_out(b):
            pltpu.make_async_copy(
                tr_v.at[b], out_hbm.at[0, :, pl.ds(0, _CB)], sem_o.at[b]
            ).wait()

        def half(g, b, first_group=False, last_group=False):
            bn = (b + 1) % _NBUF
            bi = (b + 2) % _NBUF
            if not (first_group and b < _NBUF - 1):
                wait_out(bn)
            if not (last_group and b == _NBUF - 1):
                wait_idx(bn)
                launch_gather(bn)
            wait_gather(b)
            if not (last_group and b >= _NBUF - 2):
                launch_idx(g + 2, bi)
            rows = rows_v.at[b]
            tr = tr_v.at[b]
            lanes = lax.iota(jnp.int32, _L16)
            n_rb = _CB // _L16
            base_rows = [rb * _L16 + lanes for rb in range(n_rb)]
            colbs = tuple(
                col_v.at[b][pl.ds(rb * _L16, _L16)] for rb in range(n_rb)
            )

            @plsc.parallel_loop(0, _D, unroll=4, carry=colbs)
            def t_body(d, carry, rows=rows, tr=tr, base_rows=base_rows):
                for rb in range(n_rb):
                    vals = plsc.load_gather(
                        rows, [base_rows[rb], carry[rb] + d]
                    )
                    tr[d, pl.ds(rb * _L16, _L16)] = vals * _SCALE
                return carry
            l, cb0 = chunk_lb(g)
            pltpu.async_copy(
                tr_v.at[b], out_hbm.at[l, :, pl.ds(cb0, _CB)], sem_o.at[b]
            )

        cp0 = pltpu.async_copy(
            event_hbm.at[0, pl.ds(b0w, _CB)], idx_v.at[0], sem_i.at[0]
        )
        launch_idx(1, 1)
        cp0.wait()
        launch_gather(0)

        for b in range(_NBUF):
            half(b, b, first_group=True)

        def group(gi, carry):
            for b in range(_NBUF):
                half(gi * _NBUF + b, b)
            return carry

        lax.fori_loop(1, n_groups - 1, group, 0)

        for b in range(_NBUF):
            half((n_groups - 1) * _NBUF + b, b, last_group=True)

        for b in range(1, _NBUF):
            wait_out(b)

    return k


def kernel(event, table):
    n_b, n_l = event.shape
    ev_t = event.T.astype(jnp.int32)
    table_h = table.reshape(table.shape[0] // 2, 2 * _D)
    out_ldb = _emb_kernel(n_b, n_l)(ev_t, table_h)
    return jnp.transpose(out_ldb, (2, 0, 1))


# speedup vs baseline: 1.0009x; 1.0009x over previous
import functools

import jax
import jax.numpy as jnp
from jax import lax
from jax.experimental import pallas as pl
from jax.experimental.pallas import tpu as pltpu
from jax.experimental.pallas import tpu_sc as plsc

_D = 64
_SCALE = float(_D) ** 0.5
_NW = 32
_CB = 128
_NBUF = 4
_L16 = 16


@functools.lru_cache(maxsize=None)
def _emb_kernel(n_b, n_l):
    b_per_w = n_b // _NW
    blk_per_w = b_per_w // _CB
    n_chunks = n_l * blk_per_w
    n_groups = n_chunks // _NBUF
    assert n_chunks % _NBUF == 0 and n_groups >= 3
    mesh = plsc.VectorSubcoreMesh(core_axis_name="c", subcore_axis_name="s")

    @functools.partial(
        pl.kernel,
        mesh=mesh,
        out_type=jax.ShapeDtypeStruct((n_l, _D, n_b), jnp.float32),
        scratch_types=[
            pltpu.VMEM((_NBUF, _CB), jnp.int32),
            pltpu.VMEM((_NBUF, _CB), jnp.int32),
            pltpu.VMEM((_NBUF, _CB), jnp.int32),
            pltpu.VMEM((_NBUF, _CB, 2 * _D), jnp.float32),
            pltpu.VMEM((_NBUF, _D, _CB), jnp.float32),
            pltpu.SemaphoreType.DMA((_NBUF,)),
            pltpu.SemaphoreType.DMA((_NBUF,)),
            pltpu.SemaphoreType.DMA((_NBUF,)),
        ],
        compiler_params=pltpu.CompilerParams(
            use_tc_tiling_on_sc=True, needs_layout_passes=False
        ),
    )
    def k(event_hbm, table_hbm, out_hbm, idx_v, row_v, col_v, rows_v, tr_v,
          sem_i, sem_g, sem_o):
        wid = lax.axis_index("s") * 2 + lax.axis_index("c")
        b0w = wid * b_per_w

        def chunk_lb(g):
            return g // blk_per_w, b0w + (g % blk_per_w) * _CB

        def launch_idx(g, b):
            l, cb0 = chunk_lb(g)
            pltpu.async_copy(
                event_hbm.at[l, pl.ds(cb0, _CB)], idx_v.at[b], sem_i.at[b]
            )

        def launch_gather(b):
            for v in range(_CB // _L16):
                sl = pl.ds(v * _L16, _L16)
                iv = idx_v.at[b][sl]
                row_v.at[b][sl] = lax.shift_right_logical(iv, 1)
                col_v.at[b][sl] = lax.shift_left(
                    lax.bitwise_and(iv, 1), 6
                )
            pltpu.async_copy(
                table_hbm.at[row_v.at[b]], rows_v.at[b], sem_g.at[b]
            )

        def wait_idx(b):
            pltpu.make_async_copy(
                event_hbm.at[0, pl.ds(0, _CB)], idx_v.at[b], sem_i.at[b]
            ).wait()

        def wait_gather(b):
            pltpu.make_async_copy(
                table_hbm.at[pl.ds(0, _CB)], rows_v.at[b], sem_g.at[b]
            ).wait()

        def wait_out(b):
            pltpu.make_async_copy(
                tr_v.at[b], out_hbm.at[0, :, pl.ds(0, _CB)], sem_o.at[b]
            ).wait()

        def half(g, b, first_group=False, last_group=False):
            bn = (b + 1) % _NBUF
            bi = (b + 2) % _NBUF
            if not (first_group and b < _NBUF - 1):
                wait_out(bn)
            if not (last_group and b == _NBUF - 1):
                wait_idx(bn)
                launch_gather(bn)
            wait_gather(b)
            if not (last_group and b >= _NBUF - 2):
                launch_idx(g + 2, bi)
            rows = rows_v.at[b]
            tr = tr_v.at[b]
            lanes = lax.iota(jnp.int32, _L16)
            n_rb = _CB // _L16
            base_rows = [rb * _L16 + lanes for rb in range(n_rb)]
            colbs = tuple(
                col_v.at[b][pl.ds(rb * _L16, _L16)] for rb in range(n_rb)
            )

            @plsc.parallel_loop(0, _D, unroll=8, carry=colbs)
            def t_body(d, carry, rows=rows, tr=tr, base_rows=base_rows):
                for rb in range(n_rb):
                    vals = plsc.load_gather(
                        rows, [base_rows[rb], carry[rb] + d]
                    )
                    tr[d, pl.ds(rb * _L16, _L16)] = vals * _SCALE
                return carry
            l, cb0 = chunk_lb(g)
            pltpu.async_copy(
                tr_v.at[b], out_hbm.at[l, :, pl.ds(cb0, _CB)], sem_o.at[b]
            )

        cp0 = pltpu.async_copy(
            event_hbm.at[0, pl.ds(b0w, _CB)], idx_v.at[0], sem_i.at[0]
        )
        launch_idx(1, 1)
        cp0.wait()
        launch_gather(0)

        for b in range(_NBUF):
            half(b, b, first_group=True)

        def group(gi, carry):
            for b in range(_NBUF):
                half(gi * _NBUF + b, b)
            return carry

        lax.fori_loop(1, n_groups - 1, group, 0)

        for b in range(_NBUF):
            half((n_groups - 1) * _NBUF + b, b, last_group=True)

        for b in range(1, _NBUF):
            wait_out(b)

    return k


def kernel(event, table):
    n_b, n_l = event.shape
    ev_t = event.T.astype(jnp.int32)
    table_h = table.reshape(table.shape[0] // 2, 2 * _D)
    out_ldb = _emb_kernel(n_b, n_l)(ev_t, table_h)
    return jnp.transpose(out_ldb, (2, 0, 1))


# speedup vs baseline: 1.8317x; 1.8300x over previous
import functools

import jax
import jax.numpy as jnp
from jax import lax
from jax.experimental import pallas as pl
from jax.experimental.pallas import tpu as pltpu
from jax.experimental.pallas import tpu_sc as plsc

_D = 64
_SCALE = float(_D) ** 0.5
_NW = 32
_CB = 128
_NBUF = 4
_L16 = 16


@functools.lru_cache(maxsize=None)
def _emb_kernel(n_b, n_l):
    b_per_w = n_b // _NW
    blk_per_w = b_per_w // _CB
    n_chunks = n_l * blk_per_w
    n_groups = n_chunks // _NBUF
    assert n_chunks % _NBUF == 0 and n_groups >= 3
    mesh = plsc.VectorSubcoreMesh(core_axis_name="c", subcore_axis_name="s")

    @functools.partial(
        pl.kernel,
        mesh=mesh,
        out_type=jax.ShapeDtypeStruct((n_l, _D, n_b), jnp.float32),
        scratch_types=[
            pltpu.VMEM((_NBUF, _CB), jnp.int32),
            pltpu.VMEM((_NBUF, _CB), jnp.int32),
            pltpu.VMEM((_NBUF, _CB), jnp.int32),
            pltpu.VMEM((_NBUF, _CB, 2 * _D), jnp.float32),
            pltpu.VMEM((_NBUF, _D, _CB), jnp.float32),
            pltpu.SemaphoreType.DMA((_NBUF,)),
            pltpu.SemaphoreType.DMA((_NBUF,)),
            pltpu.SemaphoreType.DMA((_NBUF,)),
        ],
        compiler_params=pltpu.CompilerParams(
            use_tc_tiling_on_sc=True, needs_layout_passes=False
        ),
    )
    def k(event_hbm, table_hbm, out_hbm, idx_v, row_v, col_v, rows_v, tr_v,
          sem_i, sem_g, sem_o):
        wid = lax.axis_index("s") * 2 + lax.axis_index("c")
        b0w = wid * b_per_w

        def chunk_lb(g):
            return g // blk_per_w, b0w + (g % blk_per_w) * _CB

        def launch_idx(g, b):
            l, cb0 = chunk_lb(g)
            pltpu.async_copy(
                event_hbm.at[l, pl.ds(cb0, _CB)], idx_v.at[b], sem_i.at[b]
            )

        def launch_gather(b):
            for v in range(_CB // _L16):
                sl = pl.ds(v * _L16, _L16)
                iv = idx_v.at[b][sl]
                row_v.at[b][sl] = lax.shift_right_logical(iv, 1)
                col_v.at[b][sl] = lax.shift_left(
                    lax.bitwise_and(iv, 1), 6
                )
            pltpu.async_copy(
                table_hbm.at[row_v.at[b]], rows_v.at[b], sem_g.at[b]
            )

        def wait_idx(b):
            pltpu.make_async_copy(
                event_hbm.at[0, pl.ds(0, _CB)], idx_v.at[b], sem_i.at[b]
            ).wait()

        def wait_gather(b):
            pltpu.make_async_copy(
                table_hbm.at[pl.ds(0, _CB)], rows_v.at[b], sem_g.at[b]
            ).wait()

        def wait_out(b):
            pltpu.make_async_copy(
                tr_v.at[b], out_hbm.at[0, :, pl.ds(0, _CB)], sem_o.at[b]
            ).wait()

        def half(g, b, first_group=False, last_group=False):
            bn = (b + 1) % _NBUF
            bi = (b + 2) % _NBUF
            if not (first_group and b < _NBUF - 1):
                wait_out(bn)
            if not (last_group and b == _NBUF - 1):
                wait_idx(bn)
                launch_gather(bn)
            wait_gather(b)
            if not (last_group and b >= _NBUF - 2):
                launch_idx(g + 2, bi)
            rows = rows_v.at[b]
            tr = tr_v.at[b]
            lanes = lax.iota(jnp.int32, _L16)
            n_rb = _CB // _L16
            base_rows = [rb * _L16 + lanes for rb in range(n_rb)]
            colbs = tuple(
                col_v.at[b][pl.ds(rb * _L16, _L16)] for rb in range(n_rb)
            )

            @plsc.parallel_loop(0, _D, unroll=4, carry=colbs)
            def t_body(k, carry, rows=rows, tr=tr, base_rows=base_rows,
                       lanes=lanes):
                diag = lax.bitwise_and(lanes + k, _D - 1)
                for rb in range(n_rb):
                    vals = plsc.load_gather(
                        rows, [base_rows[rb], carry[rb] + diag]
                    )
                    plsc.store_scatter(
                        tr, [diag, base_rows[rb]], vals * _SCALE
                    )
                return carry
            l, cb0 = chunk_lb(g)
            pltpu.async_copy(
                tr_v.at[b], out_hbm.at[l, :, pl.ds(cb0, _CB)], sem_o.at[b]
            )

        cp0 = pltpu.async_copy(
            event_hbm.at[0, pl.ds(b0w, _CB)], idx_v.at[0], sem_i.at[0]
        )
        launch_idx(1, 1)
        cp0.wait()
        launch_gather(0)

        for b in range(_NBUF):
            half(b, b, first_group=True)

        def group(gi, carry):
            for b in range(_NBUF):
                half(gi * _NBUF + b, b)
            return carry

        lax.fori_loop(1, n_groups - 1, group, 0)

        for b in range(_NBUF):
            half((n_groups - 1) * _NBUF + b, b, last_group=True)

        for b in range(1, _NBUF):
            wait_out(b)

    return k


def kernel(event, table):
    n_b, n_l = event.shape
    ev_t = event.T.astype(jnp.int32)
    table_h = table.reshape(table.shape[0] // 2, 2 * _D)
    out_ldb = _emb_kernel(n_b, n_l)(ev_t, table_h)
    return jnp.transpose(out_ldb, (2, 0, 1))


# speedup vs baseline: 2.0956x; 1.1441x over previous
import functools

import jax
import jax.numpy as jnp
from jax import lax
from jax.experimental import pallas as pl
from jax.experimental.pallas import tpu as pltpu
from jax.experimental.pallas import tpu_sc as plsc

_D = 64
_SCALE = float(_D) ** 0.5
_NW = 32
_CB = 128
_NBUF = 4
_L16 = 16


@functools.lru_cache(maxsize=None)
def _emb_kernel(n_b, n_l):
    b_per_w = n_b // _NW
    blk_per_w = b_per_w // _CB
    n_chunks = n_l * blk_per_w
    n_groups = n_chunks // _NBUF
    assert n_chunks % _NBUF == 0 and n_groups >= 3
    mesh = plsc.VectorSubcoreMesh(core_axis_name="c", subcore_axis_name="s")

    @functools.partial(
        pl.kernel,
        mesh=mesh,
        out_type=jax.ShapeDtypeStruct((n_l, _D, n_b), jnp.float32),
        scratch_types=[
            pltpu.VMEM((_NBUF, _CB), jnp.int32),
            pltpu.VMEM((_NBUF, _CB, 2 * _D), jnp.float32),
            pltpu.VMEM((_NBUF, _D, _CB), jnp.float32),
            pltpu.SemaphoreType.DMA((_NBUF,)),
            pltpu.SemaphoreType.DMA((_NBUF,)),
            pltpu.SemaphoreType.DMA((_NBUF,)),
        ],
        compiler_params=pltpu.CompilerParams(
            use_tc_tiling_on_sc=True, needs_layout_passes=False
        ),
    )
    def k(event_hbm, table_hbm, out_hbm, idx_v, rows_v, tr_v,
          sem_i, sem_g, sem_o):
        wid = lax.axis_index("s") * 2 + lax.axis_index("c")
        b0w = wid * b_per_w

        def chunk_lb(g):
            return g // blk_per_w, b0w + (g % blk_per_w) * _CB

        def launch_idx(g, b):
            l, cb0 = chunk_lb(g)
            pltpu.async_copy(
                event_hbm.at[l, pl.ds(cb0, _CB)], idx_v.at[b], sem_i.at[b]
            )

        def launch_gather(b):
            pltpu.async_copy(
                table_hbm.at[idx_v.at[b]], rows_v.at[b], sem_g.at[b]
            )

        def wait_idx(b):
            pltpu.make_async_copy(
                event_hbm.at[0, pl.ds(0, _CB)], idx_v.at[b], sem_i.at[b]
            ).wait()

        def wait_gather(b):
            pltpu.make_async_copy(
                table_hbm.at[pl.ds(0, _CB)], rows_v.at[b], sem_g.at[b]
            ).wait()

        def wait_out(b):
            pltpu.make_async_copy(
                tr_v.at[b], out_hbm.at[0, :, pl.ds(0, _CB)], sem_o.at[b]
            ).wait()

        def half(g, b, first_group=False, last_group=False):
            bn = (b + 1) % _NBUF
            bi = (b + 2) % _NBUF
            if not (first_group and b < _NBUF - 1):
                wait_out(bn)
            if not (last_group and b == _NBUF - 1):
                wait_idx(bn)
                launch_gather(bn)
            wait_gather(b)
            if not (last_group and b >= _NBUF - 2):
                launch_idx(g + 2, bi)
            rows = rows_v.at[b]
            tr = tr_v.at[b]
            lanes = lax.iota(jnp.int32, _L16)
            n_rb = _CB // _L16
            base_rows = [rb * _L16 + lanes for rb in range(n_rb)]

            @plsc.parallel_loop(0, _D, unroll=4)
            def t_body(k, rows=rows, tr=tr, base_rows=base_rows,
                       lanes=lanes):
                diag = lax.bitwise_and(lanes + k, _D - 1)
                for rb in range(n_rb):
                    vals = plsc.load_gather(
                        rows, [base_rows[rb], diag]
                    )
                    plsc.store_scatter(
                        tr, [diag, base_rows[rb]], vals * _SCALE
                    )
            l, cb0 = chunk_lb(g)
            pltpu.async_copy(
                tr_v.at[b], out_hbm.at[l, :, pl.ds(cb0, _CB)], sem_o.at[b]
            )

        cp0 = pltpu.async_copy(
            event_hbm.at[0, pl.ds(b0w, _CB)], idx_v.at[0], sem_i.at[0]
        )
        launch_idx(1, 1)
        cp0.wait()
        launch_gather(0)

        for b in range(_NBUF):
            half(b, b, first_group=True)

        def group(gi, carry):
            for b in range(_NBUF):
                half(gi * _NBUF + b, b)
            return carry

        lax.fori_loop(1, n_groups - 1, group, 0)

        for b in range(_NBUF):
            half((n_groups - 1) * _NBUF + b, b, last_group=True)

        for b in range(1, _NBUF):
            wait_out(b)

    return k


def kernel(event, table):
    n_b, n_l = event.shape
    ev_t = event.T.astype(jnp.int32)
    table_h = jnp.pad(table, ((0, 0), (0, _D)))
    out_ldb = _emb_kernel(n_b, n_l)(ev_t, table_h)
    return jnp.transpose(out_ldb, (2, 0, 1))
